# baseline (speedup 1.0000x reference)
import functools

import jax
import jax.numpy as jnp
from jax import lax
from jax.experimental import pallas as pl
from jax.experimental.pallas import tpu as pltpu
from jax.experimental.pallas import tpu_sc as plsc

N = 10000
E = 320000
NUM_GRAPHS = 64
NP1 = N + 1
NP2 = 10112
ROWS_PER_TILE = NP2 // 16
K = 128
NW = 32
CHUNKS = 80
PER_W = CHUNKS * K
E_PAD = NW * PER_W
IDX_ROWS = NW * CHUNKS + 2
E_ALLOC = IDX_ROWS * K

_PIECES = ((0, 128), (128, 128), (256, 128), (384, 128), (512, 120))


def _seg_sum_sc(h_pad, src_1d, dst_1d, zrows):
    d = h_pad.shape[1]
    mesh = plsc.VectorSubcoreMesh(core_axis_name="c", subcore_axis_name="s")

    @functools.partial(
        pl.kernel,
        out_type=jax.ShapeDtypeStruct((2 * NP2, d), jnp.float32),
        mesh=mesh,
        scratch_types=[
            pltpu.VMEM((K,), jnp.int32),
            pltpu.VMEM((K,), jnp.int32),
            pltpu.VMEM((K, d), jnp.float32),
            pltpu.VMEM_SHARED((NP2, d), jnp.float32),
            pltpu.SemaphoreType.DMA,
        ],
        compiler_params=pltpu.CompilerParams(use_tc_tiling_on_sc=False),
    )
    def k(h_hbm, src_hbm, dst_hbm, z_hbm, out_hbm, src_v, dst_v, rows_v,
          agg_sh, sem):
        c = lax.axis_index("c")
        s = lax.axis_index("s")
        r0 = s * ROWS_PER_TILE

        for t, sz in _PIECES:
            pltpu.sync_copy(z_hbm.at[pl.ds(0, sz)],
                            agg_sh.at[pl.ds(r0 + t, sz)])
        plsc.subcore_barrier()

        base = (c * 16 + s) * PER_W

        def body(j, carry):
            off = pl.multiple_of(base + j * K, K)
            pltpu.sync_copy(src_hbm.at[pl.ds(off, K)], src_v)
            pltpu.sync_copy(dst_hbm.at[pl.ds(off, K)], dst_v)
            pltpu.async_copy(h_hbm.at[src_v], rows_v, sem).wait()
            pltpu.sync_copy(rows_v, agg_sh.at[dst_v], add=True)
            return carry

        lax.fori_loop(0, CHUNKS, body, 0)
        plsc.subcore_barrier()

        out0 = c * NP2 + r0
        for t, sz in _PIECES:
            pltpu.sync_copy(agg_sh.at[pl.ds(r0 + t, sz)],
                            out_hbm.at[pl.ds(out0 + t, sz)])

    return k(h_pad, src_1d, dst_1d, zrows)


def _layer_tc(h, aggs, W1, b1, W2, b2, g, be, relu_out):
    n, din = h.shape
    dout = W1.shape[1]

    def body(h_ref, agg_ref, w1_ref, b1_ref, w2_ref, b2_ref, g_ref, be_ref,
             o_ref):
        m = h_ref[...] + agg_ref[0:N, :] + agg_ref[NP2:NP2 + N, :]
        a = jnp.dot(m, w1_ref[...], preferred_element_type=jnp.float32)
        a = jnp.maximum(a + b1_ref[...], 0.0)
        t = jnp.dot(a, w2_ref[...], preferred_element_type=jnp.float32)
        t = t + b2_ref[...]
        mu = jnp.mean(t, axis=0, keepdims=True)
        var = jnp.mean((t - mu) ** 2, axis=0, keepdims=True)
        hn = (t - mu) / jnp.sqrt(var + 1e-5) * g_ref[...] + be_ref[...]
        if relu_out:
            hn = jnp.maximum(hn, 0.0)
        o_ref[...] = hn

    return pl.pallas_call(
        body,
        out_shape=jax.ShapeDtypeStruct((n, dout), jnp.float32),
    )(h, aggs, W1, b1.reshape(1, dout), W2, b2.reshape(1, dout),
      g.reshape(1, dout), be.reshape(1, dout))


def _softmax_tc(h3, batch2d, w_row, b_lin):
    n = h3.shape[0]

    def body(h_ref, b_ref, w_ref, bl_ref, o_ref):
        z = jnp.sum(h_ref[...] * w_ref[...], axis=1, keepdims=True)
        z = (z + bl_ref[...]) / 5.0
        gid = lax.broadcasted_iota(jnp.int32, (n, NUM_GRAPHS), 1)
        oh = b_ref[...] == gid
        zb = jnp.where(oh, z, -jnp.inf)
        seg_max = jnp.max(zb, axis=0, keepdims=True)
        seg_max = jnp.where(jnp.isfinite(seg_max), seg_max, 0.0)
        node_max = jnp.sum(jnp.where(oh, seg_max, 0.0), axis=1, keepdims=True)
        ez = jnp.exp(z - node_max)
        seg_sum = jnp.sum(jnp.where(oh, ez, 0.0), axis=0, keepdims=True)
        node_den = jnp.sum(jnp.where(oh, seg_sum, 0.0), axis=1, keepdims=True)
        o_ref[...] = ez / (node_den + 1e-16)

    return pl.pallas_call(
        body,
        out_shape=jax.ShapeDtypeStruct((n, 1), jnp.float32),
    )(h3, batch2d, w_row, b_lin.reshape(1, 1))


def kernel(x, edge_index, batch, W1_0, b1_0, W2_0, b2_0, gamma_0, beta_0,
           W1_1, b1_1, W2_1, b2_1, gamma_1, beta_1,
           W1_2, b1_2, W2_2, b2_2, gamma_2, beta_2, W_lin, b_lin):
    src = edge_index[0].astype(jnp.int32)
    dst = edge_index[1].astype(jnp.int32)
    npad = E_ALLOC - E
    src_fill = jnp.full((npad,), N, dtype=jnp.int32)
    dst_fill = (N + 1 + jnp.arange(npad, dtype=jnp.int32) % (NP2 - N - 1))
    src_all = jnp.concatenate([src, src_fill])
    dst_all = jnp.concatenate([dst, dst_fill])

    layer_params = [
        (W1_0, b1_0, W2_0, b2_0, gamma_0, beta_0),
        (W1_1, b1_1, W2_1, b2_1, gamma_1, beta_1),
        (W1_2, b1_2, W2_2, b2_2, gamma_2, beta_2),
    ]

    h = x
    for i in range(3):
        d = h.shape[1]
        h_pad = jnp.concatenate([h, jnp.zeros((1, d), jnp.float32)])
        zrows = jnp.zeros((K, d), jnp.float32)
        aggs = _seg_sum_sc(h_pad, src_all, dst_all, zrows)
        W1, b1, W2, b2, g, be = layer_params[i]
        h = _layer_tc(h, aggs, W1, b1, W2, b2, g, be, relu_out=(i != 2))

    return _softmax_tc(h, batch.astype(jnp.int32).reshape(N, 1),
                       W_lin.reshape(1, 32), b_lin)


# speedup vs baseline: 1.3773x; 1.3773x over previous
import functools

import jax
import jax.numpy as jnp
from jax import lax
from jax.experimental import pallas as pl
from jax.experimental.pallas import tpu as pltpu
from jax.experimental.pallas import tpu_sc as plsc

N = 10000
E = 320000
NUM_GRAPHS = 64
NP1 = N + 1
NP2 = 10112
ROWS_PER_TILE = NP2 // 16
K = 128
NW = 32
CHUNKS = 79
PER_W = CHUNKS * K
E_PAD = NW * PER_W
E_ALLOC = E_PAD

_PIECES = ((0, 128), (128, 128), (256, 128), (384, 128), (512, 120))


def _seg_sum_sc(h_pad, src_1d, dst_1d, zrows):
    d = h_pad.shape[1]
    mesh = plsc.VectorSubcoreMesh(core_axis_name="c", subcore_axis_name="s")

    @functools.partial(
        pl.kernel,
        out_type=jax.ShapeDtypeStruct((2 * NP2, d), jnp.float32),
        mesh=mesh,
        scratch_types=[
            pltpu.VMEM((K,), jnp.int32),
            pltpu.VMEM((K,), jnp.int32),
            pltpu.VMEM((K, d), jnp.float32),
            pltpu.VMEM_SHARED((NP2, d), jnp.float32),
            pltpu.SemaphoreType.DMA,
        ],
        compiler_params=pltpu.CompilerParams(use_tc_tiling_on_sc=False),
    )
    def k(h_hbm, src_hbm, dst_hbm, z_hbm, out_hbm, src_v, dst_v, rows_v,
          agg_sh, sem):
        c = lax.axis_index("c")
        s = lax.axis_index("s")
        r0 = s * ROWS_PER_TILE

        for t, sz in _PIECES:
            pltpu.sync_copy(z_hbm.at[pl.ds(0, sz)],
                            agg_sh.at[pl.ds(r0 + t, sz)])
        plsc.subcore_barrier()

        base = (c * 16 + s) * PER_W

        def body(j, carry):
            off = pl.multiple_of(base + j * K, K)
            pltpu.sync_copy(src_hbm.at[pl.ds(off, K)], src_v)
            pltpu.sync_copy(dst_hbm.at[pl.ds(off, K)], dst_v)
            pltpu.async_copy(h_hbm.at[src_v], rows_v, sem).wait()
            pltpu.sync_copy(rows_v, agg_sh.at[dst_v], add=True)
            return carry

        lax.fori_loop(0, CHUNKS, body, 0)
        plsc.subcore_barrier()

        out0 = c * NP2 + r0
        for t, sz in _PIECES:
            pltpu.sync_copy(agg_sh.at[pl.ds(r0 + t, sz)],
                            out_hbm.at[pl.ds(out0 + t, sz)])

    return k(h_pad, src_1d, dst_1d, zrows)


def _layer_tc(h, aggs, W1, b1, W2, b2, g, be, relu_out):
    n, din = h.shape
    dout = W1.shape[1]

    def body(h_ref, agg_ref, w1_ref, b1_ref, w2_ref, b2_ref, g_ref, be_ref,
             o_ref):
        m = h_ref[...] + agg_ref[0:N, :] + agg_ref[NP2:NP2 + N, :]
        a = jnp.dot(m, w1_ref[...], preferred_element_type=jnp.float32)
        a = jnp.maximum(a + b1_ref[...], 0.0)
        t = jnp.dot(a, w2_ref[...], preferred_element_type=jnp.float32)
        t = t + b2_ref[...]
        mu = jnp.mean(t, axis=0, keepdims=True)
        var = jnp.mean((t - mu) ** 2, axis=0, keepdims=True)
        hn = (t - mu) / jnp.sqrt(var + 1e-5) * g_ref[...] + be_ref[...]
        if relu_out:
            hn = jnp.maximum(hn, 0.0)
        o_ref[...] = hn

    return pl.pallas_call(
        body,
        out_shape=jax.ShapeDtypeStruct((n, dout), jnp.float32),
    )(h, aggs, W1, b1.reshape(1, dout), W2, b2.reshape(1, dout),
      g.reshape(1, dout), be.reshape(1, dout))


def _softmax_tc(h3, batch2d, w_row, b_lin):
    n = h3.shape[0]

    def body(h_ref, b_ref, w_ref, bl_ref, o_ref):
        z = jnp.sum(h_ref[...] * w_ref[...], axis=1, keepdims=True)
        z = (z + bl_ref[...]) / 5.0
        gid = lax.broadcasted_iota(jnp.int32, (n, NUM_GRAPHS), 1)
        oh = b_ref[...] == gid
        zb = jnp.where(oh, z, -jnp.inf)
        seg_max = jnp.max(zb, axis=0, keepdims=True)
        seg_max = jnp.where(jnp.isfinite(seg_max), seg_max, 0.0)
        node_max = jnp.sum(jnp.where(oh, seg_max, 0.0), axis=1, keepdims=True)
        ez = jnp.exp(z - node_max)
        seg_sum = jnp.sum(jnp.where(oh, ez, 0.0), axis=0, keepdims=True)
        node_den = jnp.sum(jnp.where(oh, seg_sum, 0.0), axis=1, keepdims=True)
        o_ref[...] = ez / (node_den + 1e-16)

    return pl.pallas_call(
        body,
        out_shape=jax.ShapeDtypeStruct((n, 1), jnp.float32),
    )(h3, batch2d, w_row, b_lin.reshape(1, 1))


def kernel(x, edge_index, batch, W1_0, b1_0, W2_0, b2_0, gamma_0, beta_0,
           W1_1, b1_1, W2_1, b2_1, gamma_1, beta_1,
           W1_2, b1_2, W2_2, b2_2, gamma_2, beta_2, W_lin, b_lin):
    src = edge_index[0].astype(jnp.int32)
    dst = edge_index[1].astype(jnp.int32)
    npad = E_ALLOC - E
    src_fill = jnp.full((npad,), N, dtype=jnp.int32)
    dst_fill = (N + 1 + jnp.arange(npad, dtype=jnp.int32) % (NP2 - N - 1))
    src_all = jnp.concatenate([src, src_fill])
    dst_all = jnp.concatenate([dst, dst_fill])

    layer_params = [
        (W1_0, b1_0, W2_0, b2_0, gamma_0, beta_0),
        (W1_1, b1_1, W2_1, b2_1, gamma_1, beta_1),
        (W1_2, b1_2, W2_2, b2_2, gamma_2, beta_2),
    ]

    h = x
    for i in range(3):
        d = h.shape[1]
        h_pad = jnp.concatenate([h, jnp.zeros((1, d), jnp.float32)])
        zrows = jnp.zeros((K, d), jnp.float32)
        aggs = _seg_sum_sc(h_pad, src_all, dst_all, zrows)
        W1, b1, W2, b2, g, be = layer_params[i]
        h = _layer_tc(h, aggs, W1, b1, W2, b2, g, be, relu_out=(i != 2))

    return _softmax_tc(h, batch.astype(jnp.int32).reshape(N, 1),
                       W_lin.reshape(1, 32), b_lin)


# speedup vs baseline: 1.9933x; 1.4472x over previous
import functools

import jax
import jax.numpy as jnp
from jax import lax
from jax.experimental import pallas as pl
from jax.experimental.pallas import tpu as pltpu
from jax.experimental.pallas import tpu_sc as plsc

N = 10000
E = 320000
NUM_GRAPHS = 64
NP1 = N + 1
NP2 = 10112
ROWS_PER_TILE = NP2 // 16
K = 128
NW = 32
CHUNKS = 79
PER_W = CHUNKS * K
E_PAD = NW * PER_W
E_ALLOC = E_PAD + K

_PIECES = ((0, 128), (128, 128), (256, 128), (384, 128), (512, 120))


def _seg_sum_sc(h_pad, src_1d, dst_1d, zrows):
    d = h_pad.shape[1]
    mesh = plsc.VectorSubcoreMesh(core_axis_name="c", subcore_axis_name="s")

    @functools.partial(
        pl.kernel,
        out_type=jax.ShapeDtypeStruct((2 * NP2, d), jnp.float32),
        mesh=mesh,
        scratch_types=[
            pltpu.VMEM((K,), jnp.int32),
            pltpu.VMEM((K,), jnp.int32),
            pltpu.VMEM((K,), jnp.int32),
            pltpu.VMEM((K,), jnp.int32),
            pltpu.VMEM((K, d), jnp.float32),
            pltpu.VMEM((K, d), jnp.float32),
            pltpu.VMEM_SHARED((NP2, d), jnp.float32),
            pltpu.SemaphoreType.DMA,
            pltpu.SemaphoreType.DMA,
            pltpu.SemaphoreType.DMA,
            pltpu.SemaphoreType.DMA,
        ],
        compiler_params=pltpu.CompilerParams(use_tc_tiling_on_sc=False),
    )
    def k(h_hbm, src_hbm, dst_hbm, z_hbm, out_hbm, sa, da, sb, db,
          rows_a, rows_b, agg_sh, si_a, si_b, sg_a, sg_b):
        c = lax.axis_index("c")
        s = lax.axis_index("s")
        r0 = s * ROWS_PER_TILE

        for t, sz in _PIECES:
            pltpu.sync_copy(z_hbm.at[pl.ds(0, sz)],
                            agg_sh.at[pl.ds(r0 + t, sz)])
        plsc.subcore_barrier()

        base = (c * 16 + s) * PER_W

        def _off(j):
            return pl.multiple_of(base + j * K, K)

        def _idx_start(j, sref, dref, sem):
            pltpu.async_copy(src_hbm.at[pl.ds(_off(j), K)], sref, sem)
            pltpu.async_copy(dst_hbm.at[pl.ds(_off(j), K)], dref, sem)

        def _idx_wait(sref, dref, sem):
            pltpu.make_async_copy(src_hbm.at[pl.ds(0, K)], sref, sem).wait()
            pltpu.make_async_copy(dst_hbm.at[pl.ds(0, K)], dref, sem).wait()

        pltpu.sync_copy(src_hbm.at[pl.ds(_off(0), K)], sa)
        pltpu.sync_copy(dst_hbm.at[pl.ds(_off(0), K)], da)
        pltpu.async_copy(h_hbm.at[sa], rows_a, sg_a)
        _idx_start(1, sb, db, si_b)

        def pair(i, carry):
            ja = 2 * i
            _idx_wait(sb, db, si_b)
            pltpu.async_copy(h_hbm.at[sb], rows_b, sg_b)
            pltpu.make_async_copy(h_hbm.at[sa], rows_a, sg_a).wait()
            pltpu.sync_copy(rows_a, agg_sh.at[da], add=True)
            _idx_start(ja + 2, sa, da, si_a)
            _idx_wait(sa, da, si_a)
            pltpu.async_copy(h_hbm.at[sa], rows_a, sg_a)
            pltpu.make_async_copy(h_hbm.at[sb], rows_b, sg_b).wait()
            pltpu.sync_copy(rows_b, agg_sh.at[db], add=True)
            _idx_start(ja + 3, sb, db, si_b)
            return carry

        lax.fori_loop(0, (CHUNKS - 1) // 2, pair, 0)
        pltpu.make_async_copy(h_hbm.at[sa], rows_a, sg_a).wait()
        pltpu.sync_copy(rows_a, agg_sh.at[da], add=True)
        _idx_wait(sb, db, si_b)
        plsc.subcore_barrier()

        out0 = c * NP2 + r0
        for t, sz in _PIECES:
            pltpu.sync_copy(agg_sh.at[pl.ds(r0 + t, sz)],
                            out_hbm.at[pl.ds(out0 + t, sz)])

    return k(h_pad, src_1d, dst_1d, zrows)


def _layer_tc(h, aggs, W1, b1, W2, b2, g, be, relu_out):
    n, din = h.shape
    dout = W1.shape[1]

    def body(h_ref, agg_ref, w1_ref, b1_ref, w2_ref, b2_ref, g_ref, be_ref,
             o_ref):
        m = h_ref[...] + agg_ref[0:N, :] + agg_ref[NP2:NP2 + N, :]
        a = jnp.dot(m, w1_ref[...], preferred_element_type=jnp.float32)
        a = jnp.maximum(a + b1_ref[...], 0.0)
        t = jnp.dot(a, w2_ref[...], preferred_element_type=jnp.float32)
        t = t + b2_ref[...]
        mu = jnp.mean(t, axis=0, keepdims=True)
        var = jnp.mean((t - mu) ** 2, axis=0, keepdims=True)
        hn = (t - mu) / jnp.sqrt(var + 1e-5) * g_ref[...] + be_ref[...]
        if relu_out:
            hn = jnp.maximum(hn, 0.0)
        o_ref[...] = hn

    return pl.pallas_call(
        body,
        out_shape=jax.ShapeDtypeStruct((n, dout), jnp.float32),
    )(h, aggs, W1, b1.reshape(1, dout), W2, b2.reshape(1, dout),
      g.reshape(1, dout), be.reshape(1, dout))


def _softmax_tc(h3, batch2d, w_row, b_lin):
    n = h3.shape[0]

    def body(h_ref, b_ref, w_ref, bl_ref, o_ref):
        z = jnp.sum(h_ref[...] * w_ref[...], axis=1, keepdims=True)
        z = (z + bl_ref[...]) / 5.0
        gid = lax.broadcasted_iota(jnp.int32, (n, NUM_GRAPHS), 1)
        oh = b_ref[...] == gid
        zb = jnp.where(oh, z, -jnp.inf)
        seg_max = jnp.max(zb, axis=0, keepdims=True)
        seg_max = jnp.where(jnp.isfinite(seg_max), seg_max, 0.0)
        node_max = jnp.sum(jnp.where(oh, seg_max, 0.0), axis=1, keepdims=True)
        ez = jnp.exp(z - node_max)
        seg_sum = jnp.sum(jnp.where(oh, ez, 0.0), axis=0, keepdims=True)
        node_den = jnp.sum(jnp.where(oh, seg_sum, 0.0), axis=1, keepdims=True)
        o_ref[...] = ez / (node_den + 1e-16)

    return pl.pallas_call(
        body,
        out_shape=jax.ShapeDtypeStruct((n, 1), jnp.float32),
    )(h3, batch2d, w_row, b_lin.reshape(1, 1))


def kernel(x, edge_index, batch, W1_0, b1_0, W2_0, b2_0, gamma_0, beta_0,
           W1_1, b1_1, W2_1, b2_1, gamma_1, beta_1,
           W1_2, b1_2, W2_2, b2_2, gamma_2, beta_2, W_lin, b_lin):
    src = edge_index[0].astype(jnp.int32)
    dst = edge_index[1].astype(jnp.int32)
    npad = E_ALLOC - E
    src_fill = jnp.full((npad,), N, dtype=jnp.int32)
    dst_fill = (N + 1 + jnp.arange(npad, dtype=jnp.int32) % (NP2 - N - 1))
    src_all = jnp.concatenate([src, src_fill])
    dst_all = jnp.concatenate([dst, dst_fill])

    layer_params = [
        (W1_0, b1_0, W2_0, b2_0, gamma_0, beta_0),
        (W1_1, b1_1, W2_1, b2_1, gamma_1, beta_1),
        (W1_2, b1_2, W2_2, b2_2, gamma_2, beta_2),
    ]

    h = x
    for i in range(3):
        d = h.shape[1]
        h_pad = jnp.concatenate([h, jnp.zeros((1, d), jnp.float32)])
        zrows = jnp.zeros((K, d), jnp.float32)
        aggs = _seg_sum_sc(h_pad, src_all, dst_all, zrows)
        W1, b1, W2, b2, g, be = layer_params[i]
        h = _layer_tc(h, aggs, W1, b1, W2, b2, g, be, relu_out=(i != 2))

    return _softmax_tc(h, batch.astype(jnp.int32).reshape(N, 1),
                       W_lin.reshape(1, 32), b_lin)


# speedup vs baseline: 2.2024x; 1.1049x over previous
import functools

import jax
import jax.numpy as jnp
from jax import lax
from jax.experimental import pallas as pl
from jax.experimental.pallas import tpu as pltpu
from jax.experimental.pallas import tpu_sc as plsc

N = 10000
E = 320000
NUM_GRAPHS = 64
NP1 = N + 1
NP2 = 10112
ROWS_PER_TILE = NP2 // 16
K = 128
NW = 32
CH_TOT = 158
E_PAD = 16 * CH_TOT * K
E_ALLOC = E_PAD + K
CH_SPLIT_128 = (107, 51)
CH_SPLIT_64 = (97, 61)

_PIECES = ((0, 128), (128, 128), (256, 128), (384, 128), (512, 120))


def _seg_sum_sc(h_pad, src_1d, dst_1d, zrows, ch_split):
    d = h_pad.shape[1]
    ch0, ch1 = ch_split
    mesh = plsc.VectorSubcoreMesh(core_axis_name="c", subcore_axis_name="s")

    @functools.partial(
        pl.kernel,
        out_type=jax.ShapeDtypeStruct((2 * NP2, d), jnp.float32),
        mesh=mesh,
        scratch_types=[
            pltpu.VMEM((K,), jnp.int32),
            pltpu.VMEM((K,), jnp.int32),
            pltpu.VMEM((K,), jnp.int32),
            pltpu.VMEM((K,), jnp.int32),
            pltpu.VMEM((K, d), jnp.float32),
            pltpu.VMEM((K, d), jnp.float32),
            pltpu.VMEM_SHARED((NP2, d), jnp.float32),
            pltpu.SemaphoreType.DMA,
            pltpu.SemaphoreType.DMA,
            pltpu.SemaphoreType.DMA,
            pltpu.SemaphoreType.DMA,
        ],
        compiler_params=pltpu.CompilerParams(use_tc_tiling_on_sc=False),
    )
    def k(h_hbm, src_hbm, dst_hbm, z_hbm, out_hbm, sa, da, sb, db,
          rows_a, rows_b, agg_sh, si_a, si_b, sg_a, sg_b):
        c = lax.axis_index("c")
        s = lax.axis_index("s")
        r0 = s * ROWS_PER_TILE

        for t, sz in _PIECES:
            pltpu.sync_copy(z_hbm.at[pl.ds(0, sz)],
                            agg_sh.at[pl.ds(r0 + t, sz)])
        plsc.subcore_barrier()

        base = jnp.where(c == 0, s * (ch0 * K),
                         16 * (ch0 * K) + s * (ch1 * K))
        npair = jnp.where(c == 0, (ch0 - 1) // 2, (ch1 - 1) // 2)

        def _off(j):
            return pl.multiple_of(base + j * K, K)

        def _idx_start(j, sref, dref, sem):
            pltpu.async_copy(src_hbm.at[pl.ds(_off(j), K)], sref, sem)
            pltpu.async_copy(dst_hbm.at[pl.ds(_off(j), K)], dref, sem)

        def _idx_wait(sref, dref, sem):
            pltpu.make_async_copy(src_hbm.at[pl.ds(0, K)], sref, sem).wait()
            pltpu.make_async_copy(dst_hbm.at[pl.ds(0, K)], dref, sem).wait()

        pltpu.sync_copy(src_hbm.at[pl.ds(_off(0), K)], sa)
        pltpu.sync_copy(dst_hbm.at[pl.ds(_off(0), K)], da)
        pltpu.async_copy(h_hbm.at[sa], rows_a, sg_a)
        _idx_start(1, sb, db, si_b)

        def pair(i, carry):
            ja = 2 * i
            _idx_wait(sb, db, si_b)
            pltpu.async_copy(h_hbm.at[sb], rows_b, sg_b)
            pltpu.make_async_copy(h_hbm.at[sa], rows_a, sg_a).wait()
            pltpu.sync_copy(rows_a, agg_sh.at[da], add=True)
            _idx_start(ja + 2, sa, da, si_a)
            _idx_wait(sa, da, si_a)
            pltpu.async_copy(h_hbm.at[sa], rows_a, sg_a)
            pltpu.make_async_copy(h_hbm.at[sb], rows_b, sg_b).wait()
            pltpu.sync_copy(rows_b, agg_sh.at[db], add=True)
            _idx_start(ja + 3, sb, db, si_b)
            return carry

        lax.fori_loop(0, npair, pair, 0)
        pltpu.make_async_copy(h_hbm.at[sa], rows_a, sg_a).wait()
        pltpu.sync_copy(rows_a, agg_sh.at[da], add=True)
        _idx_wait(sb, db, si_b)
        plsc.subcore_barrier()

        out0 = c * NP2 + r0
        for t, sz in _PIECES:
            pltpu.sync_copy(agg_sh.at[pl.ds(r0 + t, sz)],
                            out_hbm.at[pl.ds(out0 + t, sz)])

    return k(h_pad, src_1d, dst_1d, zrows)


def _layer_tc(h, aggs, W1, b1, W2, b2, g, be, relu_out):
    n, din = h.shape
    dout = W1.shape[1]

    def body(h_ref, agg_ref, w1_ref, b1_ref, w2_ref, b2_ref, g_ref, be_ref,
             o_ref):
        m = h_ref[...] + agg_ref[0:N, :] + agg_ref[NP2:NP2 + N, :]
        a = jnp.dot(m, w1_ref[...], preferred_element_type=jnp.float32)
        a = jnp.maximum(a + b1_ref[...], 0.0)
        t = jnp.dot(a, w2_ref[...], preferred_element_type=jnp.float32)
        t = t + b2_ref[...]
        mu = jnp.mean(t, axis=0, keepdims=True)
        var = jnp.mean((t - mu) ** 2, axis=0, keepdims=True)
        hn = (t - mu) / jnp.sqrt(var + 1e-5) * g_ref[...] + be_ref[...]
        if relu_out:
            hn = jnp.maximum(hn, 0.0)
        o_ref[...] = hn

    return pl.pallas_call(
        body,
        out_shape=jax.ShapeDtypeStruct((n, dout), jnp.float32),
    )(h, aggs, W1, b1.reshape(1, dout), W2, b2.reshape(1, dout),
      g.reshape(1, dout), be.reshape(1, dout))


def _softmax_tc(h3, batch2d, w_row, b_lin):
    n = h3.shape[0]

    def body(h_ref, b_ref, w_ref, bl_ref, o_ref):
        z = jnp.sum(h_ref[...] * w_ref[...], axis=1, keepdims=True)
        z = (z + bl_ref[...]) / 5.0
        gid = lax.broadcasted_iota(jnp.int32, (n, NUM_GRAPHS), 1)
        oh = b_ref[...] == gid
        zb = jnp.where(oh, z, -jnp.inf)
        seg_max = jnp.max(zb, axis=0, keepdims=True)
        seg_max = jnp.where(jnp.isfinite(seg_max), seg_max, 0.0)
        node_max = jnp.sum(jnp.where(oh, seg_max, 0.0), axis=1, keepdims=True)
        ez = jnp.exp(z - node_max)
        seg_sum = jnp.sum(jnp.where(oh, ez, 0.0), axis=0, keepdims=True)
        node_den = jnp.sum(jnp.where(oh, seg_sum, 0.0), axis=1, keepdims=True)
        o_ref[...] = ez / (node_den + 1e-16)

    return pl.pallas_call(
        body,
        out_shape=jax.ShapeDtypeStruct((n, 1), jnp.float32),
    )(h3, batch2d, w_row, b_lin.reshape(1, 1))


def kernel(x, edge_index, batch, W1_0, b1_0, W2_0, b2_0, gamma_0, beta_0,
           W1_1, b1_1, W2_1, b2_1, gamma_1, beta_1,
           W1_2, b1_2, W2_2, b2_2, gamma_2, beta_2, W_lin, b_lin):
    src = edge_index[0].astype(jnp.int32)
    dst = edge_index[1].astype(jnp.int32)
    npad = E_ALLOC - E
    src_fill = jnp.full((npad,), N, dtype=jnp.int32)
    dst_fill = (N + 1 + jnp.arange(npad, dtype=jnp.int32) % (NP2 - N - 1))
    src_all = jnp.concatenate([src, src_fill])
    dst_all = jnp.concatenate([dst, dst_fill])

    layer_params = [
        (W1_0, b1_0, W2_0, b2_0, gamma_0, beta_0),
        (W1_1, b1_1, W2_1, b2_1, gamma_1, beta_1),
        (W1_2, b1_2, W2_2, b2_2, gamma_2, beta_2),
    ]

    h = x
    for i in range(3):
        d = h.shape[1]
        h_pad = jnp.concatenate([h, jnp.zeros((1, d), jnp.float32)])
        zrows = jnp.zeros((K, d), jnp.float32)
        aggs = _seg_sum_sc(h_pad, src_all, dst_all, zrows,
                           CH_SPLIT_128 if d == 128 else CH_SPLIT_64)
        W1, b1, W2, b2, g, be = layer_params[i]
        h = _layer_tc(h, aggs, W1, b1, W2, b2, g, be, relu_out=(i != 2))

    return _softmax_tc(h, batch.astype(jnp.int32).reshape(N, 1),
                       W_lin.reshape(1, 32), b_lin)


# speedup vs baseline: 2.3585x; 1.0709x over previous
import functools

import jax
import jax.numpy as jnp
from jax import lax
from jax.experimental import pallas as pl
from jax.experimental.pallas import tpu as pltpu
from jax.experimental.pallas import tpu_sc as plsc

N = 10000
E = 320000
NUM_GRAPHS = 64
NP1 = N + 1
NP2 = 10112
ROWS_PER_TILE = NP2 // 16
K = 128
NW = 32
CH_TOT = 158
E_PAD = 16 * CH_TOT * K
E_ALLOC = E_PAD + K
CH_SPLIT_128 = (107, 51)
CH_SPLIT_64 = (97, 61)

_PIECES = ((0, 128), (128, 128), (256, 128), (384, 128), (512, 120))


def _seg_sum_sc(h_pad, src_1d, dst_1d, zrows, ch_split):
    d = h_pad.shape[1]
    ch0, ch1 = ch_split
    mesh = plsc.VectorSubcoreMesh(core_axis_name="c", subcore_axis_name="s")

    @functools.partial(
        pl.kernel,
        out_type=jax.ShapeDtypeStruct((2 * NP2, d), jnp.float32),
        mesh=mesh,
        scratch_types=[
            pltpu.VMEM((K,), jnp.int32),
            pltpu.VMEM((K,), jnp.int32),
            pltpu.VMEM((K,), jnp.int32),
            pltpu.VMEM((K,), jnp.int32),
            pltpu.VMEM((K, d), jnp.float32),
            pltpu.VMEM((K, d), jnp.float32),
            pltpu.VMEM_SHARED((NP2, d), jnp.float32),
            pltpu.SemaphoreType.DMA,
            pltpu.SemaphoreType.DMA,
            pltpu.SemaphoreType.DMA,
            pltpu.SemaphoreType.DMA,
        ],
        compiler_params=pltpu.CompilerParams(use_tc_tiling_on_sc=False),
    )
    def k(h_hbm, src_hbm, dst_hbm, z_hbm, out_hbm, sa, da, sb, db,
          rows_a, rows_b, agg_sh, si_a, si_b, sg_a, sg_b):
        c = lax.axis_index("c")
        s = lax.axis_index("s")
        r0 = s * ROWS_PER_TILE

        for t, sz in _PIECES:
            pltpu.sync_copy(z_hbm.at[pl.ds(0, sz)],
                            agg_sh.at[pl.ds(r0 + t, sz)])
        plsc.subcore_barrier()

        base = jnp.where(c == 0, s * (ch0 * K),
                         16 * (ch0 * K) + s * (ch1 * K))
        npair = jnp.where(c == 0, (ch0 - 1) // 2, (ch1 - 1) // 2)

        def _off(j):
            return pl.multiple_of(base + j * K, K)

        def _idx_start(j, sref, dref, sem):
            pltpu.async_copy(src_hbm.at[pl.ds(_off(j), K)], sref, sem)
            pltpu.async_copy(dst_hbm.at[pl.ds(_off(j), K)], dref, sem)

        def _idx_wait(sref, dref, sem):
            pltpu.make_async_copy(src_hbm.at[pl.ds(0, K)], sref, sem).wait()
            pltpu.make_async_copy(dst_hbm.at[pl.ds(0, K)], dref, sem).wait()

        pltpu.sync_copy(src_hbm.at[pl.ds(_off(0), K)], sa)
        pltpu.sync_copy(dst_hbm.at[pl.ds(_off(0), K)], da)
        pltpu.async_copy(h_hbm.at[sa], rows_a, sg_a)
        _idx_start(1, sb, db, si_b)

        def pair(i, carry):
            ja = 2 * i
            _idx_wait(sb, db, si_b)
            pltpu.async_copy(h_hbm.at[sb], rows_b, sg_b)
            pltpu.make_async_copy(h_hbm.at[sa], rows_a, sg_a).wait()
            pltpu.sync_copy(rows_a, agg_sh.at[da], add=True)
            _idx_start(ja + 2, sa, da, si_a)
            _idx_wait(sa, da, si_a)
            pltpu.async_copy(h_hbm.at[sa], rows_a, sg_a)
            pltpu.make_async_copy(h_hbm.at[sb], rows_b, sg_b).wait()
            pltpu.sync_copy(rows_b, agg_sh.at[db], add=True)
            _idx_start(ja + 3, sb, db, si_b)
            return carry

        lax.fori_loop(0, npair, pair, 0)
        pltpu.make_async_copy(h_hbm.at[sa], rows_a, sg_a).wait()
        pltpu.sync_copy(rows_a, agg_sh.at[da], add=True)
        _idx_wait(sb, db, si_b)
        plsc.subcore_barrier()

        out0 = c * NP2 + r0
        for t, sz in _PIECES:
            pltpu.sync_copy(agg_sh.at[pl.ds(r0 + t, sz)],
                            out_hbm.at[pl.ds(out0 + t, sz)])

    return k(h_pad, src_1d, dst_1d, zrows)


def _layer_tc(h_pad, aggs, W1, b1, W2, b2, g, be, relu_out):
    dout = W1.shape[1]

    def body(h_ref, agg_ref, w1_ref, b1_ref, w2_ref, b2_ref, g_ref, be_ref,
             o_ref):
        m = h_ref[0:N, :] + agg_ref[0:N, :] + agg_ref[NP2:NP2 + N, :]
        a = jnp.dot(m, w1_ref[...], preferred_element_type=jnp.float32)
        a = jnp.maximum(a + b1_ref[...], 0.0)
        t = jnp.dot(a, w2_ref[...], preferred_element_type=jnp.float32)
        t = t + b2_ref[...]
        mu = jnp.mean(t, axis=0, keepdims=True)
        var = jnp.mean((t - mu) ** 2, axis=0, keepdims=True)
        hn = (t - mu) / jnp.sqrt(var + 1e-5) * g_ref[...] + be_ref[...]
        if relu_out:
            hn = jnp.maximum(hn, 0.0)
        o_ref[0:N, :] = hn
        o_ref[N:NP1, :] = jnp.zeros((1, dout), jnp.float32)

    return pl.pallas_call(
        body,
        out_shape=jax.ShapeDtypeStruct((NP1, dout), jnp.float32),
    )(h_pad, aggs, W1, b1.reshape(1, dout), W2, b2.reshape(1, dout),
      g.reshape(1, dout), be.reshape(1, dout))


def _softmax_tc(h3, batch2d, w_row, b_lin):
    n = N

    def body(h_ref, b_ref, w_ref, bl_ref, o_ref):
        z = jnp.sum(h_ref[0:N, :] * w_ref[...], axis=1, keepdims=True)
        z = (z + bl_ref[...]) / 5.0
        gid = lax.broadcasted_iota(jnp.int32, (n, NUM_GRAPHS), 1)
        oh = b_ref[...] == gid
        zb = jnp.where(oh, z, -jnp.inf)
        seg_max = jnp.max(zb, axis=0, keepdims=True)
        seg_max = jnp.where(jnp.isfinite(seg_max), seg_max, 0.0)
        node_max = jnp.sum(jnp.where(oh, seg_max, 0.0), axis=1, keepdims=True)
        ez = jnp.exp(z - node_max)
        seg_sum = jnp.sum(jnp.where(oh, ez, 0.0), axis=0, keepdims=True)
        node_den = jnp.sum(jnp.where(oh, seg_sum, 0.0), axis=1, keepdims=True)
        o_ref[...] = ez / (node_den + 1e-16)

    return pl.pallas_call(
        body,
        out_shape=jax.ShapeDtypeStruct((n, 1), jnp.float32),
    )(h3, batch2d, w_row, b_lin.reshape(1, 1))


def kernel(x, edge_index, batch, W1_0, b1_0, W2_0, b2_0, gamma_0, beta_0,
           W1_1, b1_1, W2_1, b2_1, gamma_1, beta_1,
           W1_2, b1_2, W2_2, b2_2, gamma_2, beta_2, W_lin, b_lin):
    src = edge_index[0].astype(jnp.int32)
    dst = edge_index[1].astype(jnp.int32)
    npad = E_ALLOC - E
    src_fill = jnp.full((npad,), N, dtype=jnp.int32)
    dst_fill = (N + 1 + jnp.arange(npad, dtype=jnp.int32) % (NP2 - N - 1))
    src_all = jnp.concatenate([src, src_fill])
    dst_all = jnp.concatenate([dst, dst_fill])

    layer_params = [
        (W1_0, b1_0, W2_0, b2_0, gamma_0, beta_0),
        (W1_1, b1_1, W2_1, b2_1, gamma_1, beta_1),
        (W1_2, b1_2, W2_2, b2_2, gamma_2, beta_2),
    ]

    h = jnp.concatenate([x, jnp.zeros((1, x.shape[1]), jnp.float32)])
    for i in range(3):
        d = h.shape[1]
        zrows = jnp.zeros((K, d), jnp.float32)
        aggs = _seg_sum_sc(h, src_all, dst_all, zrows,
                           CH_SPLIT_128 if d == 128 else CH_SPLIT_64)
        W1, b1, W2, b2, g, be = layer_params[i]
        h = _layer_tc(h, aggs, W1, b1, W2, b2, g, be, relu_out=(i != 2))

    return _softmax_tc(h, batch.astype(jnp.int32).reshape(N, 1),
                       W_lin.reshape(1, 32), b_lin)


# speedup vs baseline: 2.4242x; 1.0278x over previous
import functools

import jax
import jax.numpy as jnp
from jax import lax
from jax.experimental import pallas as pl
from jax.experimental.pallas import tpu as pltpu
from jax.experimental.pallas import tpu_sc as plsc

N = 10000
E = 320000
NUM_GRAPHS = 64
NP1 = N + 1
NP2 = 10112
ROWS_PER_TILE = NP2 // 16
K = 128
NW = 32
CH_TOT = 158
E_PAD = 16 * CH_TOT * K
E_ALLOC = E_PAD + K
CH_SPLIT_128 = (107, 51)
CH_SPLIT_64 = (97, 61)

_PIECES = ((0, 128), (128, 128), (256, 128), (384, 128), (512, 120))


def _seg_sum_sc(h_pad, src_1d, dst_1d, zrows, ch_split):
    d = h_pad.shape[1]
    ch0, ch1 = ch_split
    mesh = plsc.VectorSubcoreMesh(core_axis_name="c", subcore_axis_name="s")

    @functools.partial(
        pl.kernel,
        out_type=jax.ShapeDtypeStruct((2 * NP2, d), jnp.float32),
        mesh=mesh,
        scratch_types=[
            pltpu.VMEM((K,), jnp.int32),
            pltpu.VMEM((K,), jnp.int32),
            pltpu.VMEM((K,), jnp.int32),
            pltpu.VMEM((K,), jnp.int32),
            pltpu.VMEM((K, d), jnp.float32),
            pltpu.VMEM((K, d), jnp.float32),
            pltpu.VMEM_SHARED((NP2, d), jnp.float32),
            pltpu.SemaphoreType.DMA,
            pltpu.SemaphoreType.DMA,
            pltpu.SemaphoreType.DMA,
            pltpu.SemaphoreType.DMA,
        ],
        compiler_params=pltpu.CompilerParams(use_tc_tiling_on_sc=False),
    )
    def k(h_hbm, src_hbm, dst_hbm, z_hbm, out_hbm, sa, da, sb, db,
          rows_a, rows_b, agg_sh, si_a, si_b, sg_a, sg_b):
        c = lax.axis_index("c")
        s = lax.axis_index("s")
        r0 = s * ROWS_PER_TILE

        pltpu.sync_copy(z_hbm.at[pl.ds(0, K)], rows_a)
        for t, sz in _PIECES:
            pltpu.sync_copy(rows_a.at[pl.ds(0, sz)],
                            agg_sh.at[pl.ds(r0 + t, sz)])
        plsc.subcore_barrier()

        base = jnp.where(c == 0, s * (ch0 * K),
                         16 * (ch0 * K) + s * (ch1 * K))
        npair = jnp.where(c == 0, (ch0 - 1) // 2, (ch1 - 1) // 2)

        def _off(j):
            return pl.multiple_of(base + j * K, K)

        def _idx_start(j, sref, dref, sem):
            pltpu.async_copy(src_hbm.at[pl.ds(_off(j), K)], sref, sem)
            pltpu.async_copy(dst_hbm.at[pl.ds(_off(j), K)], dref, sem)

        def _idx_wait(sref, dref, sem):
            pltpu.make_async_copy(src_hbm.at[pl.ds(0, K)], sref, sem).wait()
            pltpu.make_async_copy(dst_hbm.at[pl.ds(0, K)], dref, sem).wait()

        pltpu.sync_copy(src_hbm.at[pl.ds(_off(0), K)], sa)
        pltpu.sync_copy(dst_hbm.at[pl.ds(_off(0), K)], da)
        pltpu.async_copy(h_hbm.at[sa], rows_a, sg_a)
        _idx_start(1, sb, db, si_b)

        def pair(i, carry):
            ja = 2 * i
            _idx_wait(sb, db, si_b)
            pltpu.async_copy(h_hbm.at[sb], rows_b, sg_b)
            pltpu.make_async_copy(h_hbm.at[sa], rows_a, sg_a).wait()
            pltpu.sync_copy(rows_a, agg_sh.at[da], add=True)
            _idx_start(ja + 2, sa, da, si_a)
            _idx_wait(sa, da, si_a)
            pltpu.async_copy(h_hbm.at[sa], rows_a, sg_a)
            pltpu.make_async_copy(h_hbm.at[sb], rows_b, sg_b).wait()
            pltpu.sync_copy(rows_b, agg_sh.at[db], add=True)
            _idx_start(ja + 3, sb, db, si_b)
            return carry

        lax.fori_loop(0, npair, pair, 0)
        pltpu.make_async_copy(h_hbm.at[sa], rows_a, sg_a).wait()
        pltpu.sync_copy(rows_a, agg_sh.at[da], add=True)
        _idx_wait(sb, db, si_b)
        plsc.subcore_barrier()

        out0 = c * NP2 + r0
        for t, sz in _PIECES:
            pltpu.sync_copy(agg_sh.at[pl.ds(r0 + t, sz)],
                            out_hbm.at[pl.ds(out0 + t, sz)])

    return k(h_pad, src_1d, dst_1d, zrows)


def _layer_tc(h_pad, aggs, W1, b1, W2, b2, g, be, relu_out):
    dout = W1.shape[1]

    def body(h_ref, agg_ref, w1_ref, b1_ref, w2_ref, b2_ref, g_ref, be_ref,
             o_ref):
        m = h_ref[0:N, :] + agg_ref[0:N, :] + agg_ref[NP2:NP2 + N, :]
        a = jnp.dot(m, w1_ref[...], preferred_element_type=jnp.float32)
        a = jnp.maximum(a + b1_ref[...], 0.0)
        t = jnp.dot(a, w2_ref[...], preferred_element_type=jnp.float32)
        t = t + b2_ref[...]
        mu = jnp.mean(t, axis=0, keepdims=True)
        var = jnp.mean((t - mu) ** 2, axis=0, keepdims=True)
        hn = (t - mu) / jnp.sqrt(var + 1e-5) * g_ref[...] + be_ref[...]
        if relu_out:
            hn = jnp.maximum(hn, 0.0)
        o_ref[0:N, :] = hn
        o_ref[N:NP1, :] = jnp.zeros((1, dout), jnp.float32)

    return pl.pallas_call(
        body,
        out_shape=jax.ShapeDtypeStruct((NP1, dout), jnp.float32),
    )(h_pad, aggs, W1, b1.reshape(1, dout), W2, b2.reshape(1, dout),
      g.reshape(1, dout), be.reshape(1, dout))


def _softmax_tc(h3, batch2d, w_row, b_lin):
    n = N

    def body(h_ref, b_ref, w_ref, bl_ref, o_ref):
        z = jnp.sum(h_ref[0:N, :] * w_ref[...], axis=1, keepdims=True)
        z = (z + bl_ref[...]) / 5.0
        gid = lax.broadcasted_iota(jnp.int32, (n, NUM_GRAPHS), 1)
        oh = b_ref[...] == gid
        zb = jnp.where(oh, z, -jnp.inf)
        seg_max = jnp.max(zb, axis=0, keepdims=True)
        seg_max = jnp.where(jnp.isfinite(seg_max), seg_max, 0.0)
        node_max = jnp.sum(jnp.where(oh, seg_max, 0.0), axis=1, keepdims=True)
        ez = jnp.exp(z - node_max)
        seg_sum = jnp.sum(jnp.where(oh, ez, 0.0), axis=0, keepdims=True)
        node_den = jnp.sum(jnp.where(oh, seg_sum, 0.0), axis=1, keepdims=True)
        o_ref[...] = ez / (node_den + 1e-16)

    return pl.pallas_call(
        body,
        out_shape=jax.ShapeDtypeStruct((n, 1), jnp.float32),
    )(h3, batch2d, w_row, b_lin.reshape(1, 1))


def kernel(x, edge_index, batch, W1_0, b1_0, W2_0, b2_0, gamma_0, beta_0,
           W1_1, b1_1, W2_1, b2_1, gamma_1, beta_1,
           W1_2, b1_2, W2_2, b2_2, gamma_2, beta_2, W_lin, b_lin):
    src = edge_index[0].astype(jnp.int32)
    dst = edge_index[1].astype(jnp.int32)
    npad = E_ALLOC - E
    src_fill = jnp.full((npad,), N, dtype=jnp.int32)
    dst_fill = (N + 1 + jnp.arange(npad, dtype=jnp.int32) % (NP2 - N - 1))
    src_all = jnp.concatenate([src, src_fill])
    dst_all = jnp.concatenate([dst, dst_fill])

    layer_params = [
        (W1_0, b1_0, W2_0, b2_0, gamma_0, beta_0),
        (W1_1, b1_1, W2_1, b2_1, gamma_1, beta_1),
        (W1_2, b1_2, W2_2, b2_2, gamma_2, beta_2),
    ]

    h = jnp.concatenate([x, jnp.zeros((1, x.shape[1]), jnp.float32)])
    for i in range(3):
        d = h.shape[1]
        zrows = jnp.zeros((K, d), jnp.float32)
        aggs = _seg_sum_sc(h, src_all, dst_all, zrows,
                           CH_SPLIT_128 if d == 128 else CH_SPLIT_64)
        W1, b1, W2, b2, g, be = layer_params[i]
        h = _layer_tc(h, aggs, W1, b1, W2, b2, g, be, relu_out=(i != 2))

    return _softmax_tc(h, batch.astype(jnp.int32).reshape(N, 1),
                       W_lin.reshape(1, 32), b_lin)


# speedup vs baseline: 2.5728x; 1.0613x over previous
import functools

import jax
import jax.numpy as jnp
from jax import lax
from jax.experimental import pallas as pl
from jax.experimental.pallas import tpu as pltpu
from jax.experimental.pallas import tpu_sc as plsc

N = 10000
E = 320000
NUM_GRAPHS = 64
NP1 = N + 1
NP2 = 10112
ROWS_PER_TILE = NP2 // 16
K = 128
NW = 32
CH_TOT = 158
E_PAD = 16 * CH_TOT * K
E_ALLOC = E_PAD + K
CH_SPLIT_128 = (131, 27)
CH_SPLIT_64 = (97, 61)

_PIECES = ((0, 128), (128, 128), (256, 128), (384, 128), (512, 120))


def _seg_sum_sc(h_pad, src_1d, dst_1d, zrows, ch_split):
    d = h_pad.shape[1]
    ch0, ch1 = ch_split
    mesh = plsc.VectorSubcoreMesh(core_axis_name="c", subcore_axis_name="s")

    @functools.partial(
        pl.kernel,
        out_type=jax.ShapeDtypeStruct((2 * NP2, d), jnp.float32),
        mesh=mesh,
        scratch_types=[
            pltpu.VMEM((K,), jnp.int32),
            pltpu.VMEM((K,), jnp.int32),
            pltpu.VMEM((K,), jnp.int32),
            pltpu.VMEM((K,), jnp.int32),
            pltpu.VMEM((K, d), jnp.float32),
            pltpu.VMEM((K, d), jnp.float32),
            pltpu.VMEM_SHARED((NP2, d), jnp.float32),
            pltpu.SemaphoreType.DMA,
            pltpu.SemaphoreType.DMA,
            pltpu.SemaphoreType.DMA,
            pltpu.SemaphoreType.DMA,
        ],
        compiler_params=pltpu.CompilerParams(use_tc_tiling_on_sc=False),
    )
    def k(h_hbm, src_hbm, dst_hbm, z_hbm, out_hbm, sa, da, sb, db,
          rows_a, rows_b, agg_sh, si_a, si_b, sg_a, sg_b):
        c = lax.axis_index("c")
        s = lax.axis_index("s")
        r0 = s * ROWS_PER_TILE

        pltpu.sync_copy(z_hbm.at[pl.ds(0, K)], rows_a)
        for t, sz in _PIECES:
            pltpu.sync_copy(rows_a.at[pl.ds(0, sz)],
                            agg_sh.at[pl.ds(r0 + t, sz)])
        plsc.subcore_barrier()

        base = jnp.where(c == 0, s * (ch0 * K),
                         16 * (ch0 * K) + s * (ch1 * K))
        npair = jnp.where(c == 0, (ch0 - 1) // 2, (ch1 - 1) // 2)

        def _off(j):
            return pl.multiple_of(base + j * K, K)

        def _idx_start(j, sref, dref, sem):
            pltpu.async_copy(src_hbm.at[pl.ds(_off(j), K)], sref, sem)
            pltpu.async_copy(dst_hbm.at[pl.ds(_off(j), K)], dref, sem)

        def _idx_wait(sref, dref, sem):
            pltpu.make_async_copy(src_hbm.at[pl.ds(0, K)], sref, sem).wait()
            pltpu.make_async_copy(dst_hbm.at[pl.ds(0, K)], dref, sem).wait()

        pltpu.sync_copy(src_hbm.at[pl.ds(_off(0), K)], sa)
        pltpu.sync_copy(dst_hbm.at[pl.ds(_off(0), K)], da)
        pltpu.async_copy(h_hbm.at[sa], rows_a, sg_a)
        _idx_start(1, sb, db, si_b)

        def pair(i, carry):
            ja = 2 * i
            _idx_wait(sb, db, si_b)
            pltpu.async_copy(h_hbm.at[sb], rows_b, sg_b)
            pltpu.make_async_copy(h_hbm.at[sa], rows_a, sg_a).wait()
            pltpu.sync_copy(rows_a, agg_sh.at[da], add=True)
            _idx_start(ja + 2, sa, da, si_a)
            _idx_wait(sa, da, si_a)
            pltpu.async_copy(h_hbm.at[sa], rows_a, sg_a)
            pltpu.make_async_copy(h_hbm.at[sb], rows_b, sg_b).wait()
            pltpu.sync_copy(rows_b, agg_sh.at[db], add=True)
            _idx_start(ja + 3, sb, db, si_b)
            return carry

        lax.fori_loop(0, npair, pair, 0)
        pltpu.make_async_copy(h_hbm.at[sa], rows_a, sg_a).wait()
        pltpu.sync_copy(rows_a, agg_sh.at[da], add=True)
        _idx_wait(sb, db, si_b)
        plsc.subcore_barrier()

        out0 = c * NP2 + r0
        for t, sz in _PIECES:
            pltpu.sync_copy(agg_sh.at[pl.ds(r0 + t, sz)],
                            out_hbm.at[pl.ds(out0 + t, sz)])

    return k(h_pad, src_1d, dst_1d, zrows)


def _layer_tc(h_pad, aggs, W1, b1, W2, b2, g, be, relu_out):
    dout = W1.shape[1]

    def body(h_ref, agg_ref, w1_ref, b1_ref, w2_ref, b2_ref, g_ref, be_ref,
             o_ref):
        m = h_ref[0:N, :] + agg_ref[0:N, :] + agg_ref[NP2:NP2 + N, :]
        a = jnp.dot(m, w1_ref[...], preferred_element_type=jnp.float32)
        a = jnp.maximum(a + b1_ref[...], 0.0)
        t = jnp.dot(a, w2_ref[...], preferred_element_type=jnp.float32)
        t = t + b2_ref[...]
        mu = jnp.mean(t, axis=0, keepdims=True)
        var = jnp.mean((t - mu) ** 2, axis=0, keepdims=True)
        hn = (t - mu) / jnp.sqrt(var + 1e-5) * g_ref[...] + be_ref[...]
        if relu_out:
            hn = jnp.maximum(hn, 0.0)
        o_ref[0:N, :] = hn
        o_ref[N:NP1, :] = jnp.zeros((1, dout), jnp.float32)

    return pl.pallas_call(
        body,
        out_shape=jax.ShapeDtypeStruct((NP1, dout), jnp.float32),
    )(h_pad, aggs, W1, b1.reshape(1, dout), W2, b2.reshape(1, dout),
      g.reshape(1, dout), be.reshape(1, dout))


def _softmax_tc(h3, batch2d, w_row, b_lin):
    n = N

    def body(h_ref, b_ref, w_ref, bl_ref, o_ref):
        z = jnp.sum(h_ref[0:N, :] * w_ref[...], axis=1, keepdims=True)
        z = (z + bl_ref[...]) / 5.0
        gid = lax.broadcasted_iota(jnp.int32, (n, NUM_GRAPHS), 1)
        oh = b_ref[...] == gid
        zb = jnp.where(oh, z, -jnp.inf)
        seg_max = jnp.max(zb, axis=0, keepdims=True)
        seg_max = jnp.where(jnp.isfinite(seg_max), seg_max, 0.0)
        node_max = jnp.sum(jnp.where(oh, seg_max, 0.0), axis=1, keepdims=True)
        ez = jnp.exp(z - node_max)
        seg_sum = jnp.sum(jnp.where(oh, ez, 0.0), axis=0, keepdims=True)
        node_den = jnp.sum(jnp.where(oh, seg_sum, 0.0), axis=1, keepdims=True)
        o_ref[...] = ez / (node_den + 1e-16)

    return pl.pallas_call(
        body,
        out_shape=jax.ShapeDtypeStruct((n, 1), jnp.float32),
    )(h3, batch2d, w_row, b_lin.reshape(1, 1))


def kernel(x, edge_index, batch, W1_0, b1_0, W2_0, b2_0, gamma_0, beta_0,
           W1_1, b1_1, W2_1, b2_1, gamma_1, beta_1,
           W1_2, b1_2, W2_2, b2_2, gamma_2, beta_2, W_lin, b_lin):
    src = edge_index[0].astype(jnp.int32)
    dst = edge_index[1].astype(jnp.int32)
    npad = E_ALLOC - E
    src_fill = jnp.full((npad,), N, dtype=jnp.int32)
    dst_fill = (N + 1 + jnp.arange(npad, dtype=jnp.int32) % (NP2 - N - 1))
    src_all = jnp.concatenate([src, src_fill])
    dst_all = jnp.concatenate([dst, dst_fill])

    layer_params = [
        (W1_0, b1_0, W2_0, b2_0, gamma_0, beta_0),
        (W1_1, b1_1, W2_1, b2_1, gamma_1, beta_1),
        (W1_2, b1_2, W2_2, b2_2, gamma_2, beta_2),
    ]

    h = jnp.concatenate([x, jnp.zeros((1, x.shape[1]), jnp.float32)])
    for i in range(3):
        d = h.shape[1]
        zrows = jnp.zeros((K, d), jnp.float32)
        aggs = _seg_sum_sc(h, src_all, dst_all, zrows,
                           CH_SPLIT_128 if d == 128 else CH_SPLIT_64)
        W1, b1, W2, b2, g, be = layer_params[i]
        h = _layer_tc(h, aggs, W1, b1, W2, b2, g, be, relu_out=(i != 2))

    return _softmax_tc(h, batch.astype(jnp.int32).reshape(N, 1),
                       W_lin.reshape(1, 32), b_lin)
